# baseline (speedup 1.0000x reference)
import jax
import jax.numpy as jnp
from jax import lax
from jax.experimental import pallas as pl
from jax.experimental.pallas import tpu as pltpu
from jax.experimental.pallas import tpu_sc as plsc

_NQ = 256
_B = 8
_T = 8192
_QB = 128
_TW = 4096
_TB = 256
_NCHUNK = _TW // _TB
_NBUF = 3


def _sc_body(idx_hbm, out_hbm, idx_v, buf0, buf1, buf2, sem0, sem1, sem2, isem):
    nc = 2
    wid = lax.axis_index("s") * nc + lax.axis_index("c")
    b = wid // 4
    qh = (wid // 2) % 2
    th = wid % 2
    q0 = qh * _QB
    tbase = th * _TW

    idx_cp = pltpu.make_async_copy(idx_hbm.at[b, pl.ds(tbase, _TW)], idx_v, isem)
    idx_cp.start()

    zeros16 = jnp.zeros((16,), jnp.float32)
    ones16 = jnp.ones((16,), jnp.float32)
    iota16 = lax.iota(jnp.int32, 16)

    bufs = (buf0, buf1, buf2)
    sems = (sem0, sem1, sem2)

    def _zero(buf):
        def _zbody(r, carry):
            for j in range(_TB // 16):
                buf[r, pl.ds(j * 16, 16)] = zeros16
            return carry

        lax.fori_loop(0, _QB, _zbody, 0)

    def _scatter(buf, c, vals):
        for j in range(_TB // 16):
            v_idx = idx_v[pl.ds(c * _TB + j * 16, 16)]
            row = v_idx - q0
            mask = (v_idx >= q0) & (row < _QB)
            col = iota16 + (j * 16)
            plsc.store_scatter(buf, [row, col], vals, mask=mask)

    copies = [None] * _NBUF
    for c in range(_NCHUNK):
        k = c % _NBUF
        buf = bufs[k]
        if c < _NBUF:
            _zero(buf)
            if c == 0:
                idx_cp.wait()
        else:
            copies[k].wait()
            _scatter(buf, c - _NBUF, zeros16)
        _scatter(buf, c, ones16)
        cp = pltpu.make_async_copy(
            buf,
            out_hbm.at[b, pl.ds(q0, _QB), pl.ds(tbase + c * _TB, _TB)],
            sems[k],
        )
        cp.start()
        copies[k] = cp
    for cp in copies:
        cp.wait()


def kernel(in_snd_slice, quant_onehot):
    idx = in_snd_slice.astype(jnp.int32)
    sc = pl.kernel(
        _sc_body,
        mesh=plsc.VectorSubcoreMesh(core_axis_name="c", subcore_axis_name="s"),
        out_type=jax.ShapeDtypeStruct((_B, _NQ, _T), jnp.float32),
        scratch_types=[
            pltpu.VMEM((_TW,), jnp.int32),
            pltpu.VMEM((_QB, _TB), jnp.float32),
            pltpu.VMEM((_QB, _TB), jnp.float32),
            pltpu.VMEM((_QB, _TB), jnp.float32),
            pltpu.SemaphoreType.DMA,
            pltpu.SemaphoreType.DMA,
            pltpu.SemaphoreType.DMA,
            pltpu.SemaphoreType.DMA,
        ],
        compiler_params=pltpu.CompilerParams(needs_layout_passes=False),
    )
    return sc(idx)


# speedup vs baseline: 1.1308x; 1.1308x over previous
import jax
import jax.numpy as jnp
from jax import lax
from jax.experimental import pallas as pl
from jax.experimental.pallas import tpu as pltpu
from jax.experimental.pallas import tpu_sc as plsc

_NQ = 256
_B = 8
_T = 8192
_NW = 32
_WPB = _NW // _B
_TW = _T // _WPB
_TB = 128
_NCHUNK = _TW // _TB


_NBUF = 3


def _sc_body(idx_hbm, out_hbm, idx_v, buf0, buf1, buf2, sem0, sem1, sem2, isem):
    nc = 2
    wid = lax.axis_index("s") * nc + lax.axis_index("c")
    b = wid // _WPB
    tbase = (wid % _WPB) * _TW

    idx_cp = pltpu.make_async_copy(idx_hbm.at[b, pl.ds(tbase, _TW)], idx_v, isem)
    idx_cp.start()

    zeros16 = jnp.zeros((16,), jnp.float32)
    ones16 = jnp.ones((16,), jnp.float32)
    iota16 = lax.iota(jnp.int32, 16)

    bufs = (buf0, buf1, buf2)
    sems = (sem0, sem1, sem2)

    def _zero(buf):
        @plsc.parallel_loop(0, _NQ, 1, unroll=4)
        def _zbody(r):
            for j in range(_TB // 16):
                buf[r, pl.ds(j * 16, 16)] = zeros16

    def _load_idxs(c):
        return [idx_v[pl.ds(c * _TB + j * 16, 16)] for j in range(_TB // 16)]

    def _scatter(buf, idxs, vals):
        for j in range(_TB // 16):
            col = iota16 + (j * 16)
            plsc.store_scatter(buf, [idxs[j], col], vals)

    copies = [None] * _NBUF
    live_idxs = [None] * _NBUF
    for c in range(_NCHUNK):
        k = c % _NBUF
        buf = bufs[k]
        if c < _NBUF:
            _zero(buf)
            if c == 0:
                idx_cp.wait()
        else:
            copies[k].wait()
            _scatter(buf, live_idxs[k], zeros16)
        idxs = _load_idxs(c)
        _scatter(buf, idxs, ones16)
        live_idxs[k] = idxs
        cp = pltpu.make_async_copy(
            buf, out_hbm.at[b, :, pl.ds(tbase + c * _TB, _TB)], sems[k]
        )
        cp.start()
        copies[k] = cp
    for cp in copies:
        cp.wait()


def kernel(in_snd_slice, quant_onehot):
    idx = in_snd_slice.astype(jnp.int32)
    sc = pl.kernel(
        _sc_body,
        mesh=plsc.VectorSubcoreMesh(core_axis_name="c", subcore_axis_name="s"),
        out_type=jax.ShapeDtypeStruct((_B, _NQ, _T), jnp.float32),
        scratch_types=[
            pltpu.VMEM((_TW,), jnp.int32),
            pltpu.VMEM((_NQ, _TB), jnp.float32),
            pltpu.VMEM((_NQ, _TB), jnp.float32),
            pltpu.VMEM((_NQ, _TB), jnp.float32),
            pltpu.SemaphoreType.DMA,
            pltpu.SemaphoreType.DMA,
            pltpu.SemaphoreType.DMA,
            pltpu.SemaphoreType.DMA,
        ],
        compiler_params=pltpu.CompilerParams(needs_layout_passes=False),
    )
    return sc(idx)
